# baseline (speedup 1.0000x reference)
import functools
import math

import jax
import jax.numpy as jnp
from jax import lax
from jax.experimental import pallas as pl
from jax.experimental.pallas import tpu as pltpu
from jax.experimental.pallas import tpu_sc as plsc

EMB = 64
LANES = 16
CHUNK = 128
NBUF = 4


def _make_sc_gather(num_workers: int, nchunk: int, scale: float):
    mesh = plsc.VectorSubcoreMesh(core_axis_name="c", subcore_axis_name="s")
    b_per_w = nchunk * CHUNK
    assert nchunk % NBUF == 0
    nouter = nchunk // NBUF

    scratch = [pltpu.VMEM((nchunk, CHUNK), jnp.int32)]
    scratch += [pltpu.VMEM((CHUNK, EMB), jnp.float32) for _ in range(2 * NBUF)]
    scratch += [pltpu.SemaphoreType.DMA for _ in range(2 * NBUF)]

    @functools.partial(
        pl.kernel,
        mesh=mesh,
        out_type=jax.ShapeDtypeStruct((num_workers * b_per_w, EMB), jnp.float32),
        scratch_types=scratch,
        compiler_params=pltpu.CompilerParams(use_tc_tiling_on_sc=False),
    )
    def sc_embed(tokens_hbm, table_hbm, out_hbm, idx_v, *bufs):
        rows_in = bufs[0:NBUF]
        rows_out = bufs[NBUF:2 * NBUF]
        gsem = bufs[2 * NBUF:3 * NBUF]
        ssem = bufs[3 * NBUF:4 * NBUF]

        nc = lax.axis_size("c")
        wid = lax.axis_index("s") * nc + lax.axis_index("c")
        pltpu.sync_copy(tokens_hbm.at[wid], idx_v)

        for b in range(NBUF):
            pltpu.async_copy(table_hbm.at[idx_v.at[b]], rows_in[b], gsem[b])

        def outer(o, carry):
            for b in range(NBUF):
                t = o * NBUF + b

                @pl.when(o > 0)
                def _wait_store():
                    pltpu.make_async_copy(
                        rows_out[b], out_hbm.at[pl.ds(0, CHUNK)], ssem[b]
                    ).wait()

                pltpu.make_async_copy(
                    table_hbm.at[idx_v.at[b]], rows_in[b], gsem[b]
                ).wait()

                def scale_row(r, c2, b=b):
                    for v in range(EMB // LANES):
                        sl = pl.ds(v * LANES, LANES)
                        rows_out[b][r, sl] = rows_in[b][r, sl] * scale
                    return c2

                lax.fori_loop(0, CHUNK, scale_row, 0, unroll=8)

                pltpu.async_copy(
                    rows_out[b],
                    out_hbm.at[pl.ds(wid * b_per_w + t * CHUNK, CHUNK)],
                    ssem[b],
                )

                @pl.when(t + NBUF < nchunk)
                def _refill():
                    pltpu.async_copy(
                        table_hbm.at[idx_v.at[t + NBUF]], rows_in[b], gsem[b]
                    )

            return carry

        lax.fori_loop(0, nouter, outer, 0)

        for b in range(NBUF):
            pltpu.make_async_copy(
                rows_out[b], out_hbm.at[pl.ds(0, CHUNK)], ssem[b]
            ).wait()

    return sc_embed


def kernel(tokens, table):
    bsz, seq = tokens.shape
    total = bsz * seq
    num_workers = 32
    assert total % (num_workers * CHUNK) == 0
    nchunk = total // (num_workers * CHUNK)
    scale = math.sqrt(float(EMB))
    toks = tokens.reshape(num_workers, nchunk, CHUNK).astype(jnp.int32)
    out = _make_sc_gather(num_workers, nchunk, scale)(toks, table)
    return out.reshape(bsz, seq, EMB)
